# Initial kernel scaffold; baseline (speedup 1.0000x reference)
import jax
import jax.numpy as jnp
from jax.experimental import pallas as pl


def kernel(x, embed):
    raise NotImplementedError("write your pallas kernel here")


# speedup vs baseline: 1.4157x; 1.4157x over previous
import functools

import jax
import jax.numpy as jnp
from jax import lax
from jax.experimental import pallas as pl
from jax.experimental.pallas import tpu as pltpu
from jax.experimental.pallas import tpu_sc as plsc


DIM = 256
K = 8192
B = 16
N = 1024
BN = B * N

ROWS = 256


def _dist_argmax_body(x_ref, et_ref, x2_ref, y2_ref, dist_ref, ind_ref):
    x = x_ref[...]
    et = et_ref[...]
    xy = jnp.dot(x, et, preferred_element_type=jnp.float32)
    dd = x2_ref[...] + y2_ref[...] + (-2.0) * xy
    d = -jnp.sqrt(jnp.clip(dd, 0.0, None))
    dist_ref[...] = d
    m = jnp.max(d, axis=1, keepdims=True)
    iota = lax.broadcasted_iota(jnp.int32, (ROWS, K), 1)
    ind_ref[...] = jnp.min(jnp.where(d == m, iota, K), axis=1, keepdims=True)


def _dist_argmax(xf, embed_t, x2, y2):
    grid = (BN // ROWS,)
    return pl.pallas_call(
        _dist_argmax_body,
        grid=grid,
        in_specs=[
            pl.BlockSpec((ROWS, DIM), lambda i: (i, 0)),
            pl.BlockSpec((DIM, K), lambda i: (0, 0)),
            pl.BlockSpec((ROWS, 1), lambda i: (i, 0)),
            pl.BlockSpec((1, K), lambda i: (0, 0)),
        ],
        out_specs=[
            pl.BlockSpec((ROWS, K), lambda i: (i, 0)),
            pl.BlockSpec((ROWS, 1), lambda i: (i, 0)),
        ],
        out_shape=[
            jax.ShapeDtypeStruct((BN, K), jnp.float32),
            jax.ShapeDtypeStruct((BN, 1), jnp.int32),
        ],
    )(xf, embed_t, x2, y2)


_NC, _NS = 2, 16
_NW = _NC * _NS
_BPW = BN // _NW
_CHUNK = 128
_NCHUNK = _BPW // _CHUNK


@functools.lru_cache(maxsize=None)
def _make_sc_gather():
    mesh = plsc.VectorSubcoreMesh(core_axis_name="c", subcore_axis_name="s")

    @functools.partial(
        pl.kernel, mesh=mesh,
        out_type=jax.ShapeDtypeStruct((BN, DIM), jnp.float32),
        scratch_types=[
            pltpu.VMEM((_NCHUNK, _CHUNK), jnp.int32),
            pltpu.VMEM((_CHUNK, DIM), jnp.float32),
            pltpu.SemaphoreType.DMA,
        ],
    )
    def sc_gather(idx_hbm, table_hbm, out_hbm, idx_v, rows_v, sem):
        wid = lax.axis_index("s") * _NC + lax.axis_index("c")
        pltpu.sync_copy(idx_hbm.at[wid], idx_v)
        base = wid * _BPW
        for c in range(_NCHUNK):
            pltpu.async_copy(table_hbm.at[idx_v.at[c]], rows_v, sem).wait()
            pltpu.sync_copy(rows_v, out_hbm.at[pl.ds(base + c * _CHUNK, _CHUNK)])

    return sc_gather


def kernel(x, embed):
    xf = x.reshape(BN, DIM)
    table = embed[0]
    embed_t = jnp.swapaxes(table, 0, 1)
    x2 = jnp.sum(xf * xf, axis=1, keepdims=True)
    y2 = jnp.sum(table * table, axis=-1)[None, :]
    dist2d, ind2d = _dist_argmax(xf, embed_t, x2, y2)
    ind = ind2d.reshape(BN)
    idx3 = ind.reshape(_NW, _NCHUNK, _CHUNK)
    quantize = _make_sc_gather()(idx3, table)
    return (
        quantize.reshape(B, N, DIM),
        ind.reshape(B, N),
        dist2d.reshape(1, B, N, K),
    )


# speedup vs baseline: 1.4644x; 1.0344x over previous
import functools

import jax
import jax.numpy as jnp
from jax import lax
from jax.experimental import pallas as pl
from jax.experimental.pallas import tpu as pltpu
from jax.experimental.pallas import tpu_sc as plsc


DIM = 256
K = 8192
B = 16
N = 1024
BN = B * N

ROWS = 512


def _dist_argmax_body(x_ref, et_ref, x2_ref, y2_ref, dist_ref, ind_ref):
    x = x_ref[...]
    et = et_ref[...]
    xy = jnp.dot(x, et, preferred_element_type=jnp.float32)
    dd = x2_ref[...] + y2_ref[...] + (-2.0) * xy
    d = -jnp.sqrt(jnp.clip(dd, 0.0, None))
    dist_ref[...] = d
    m = jnp.max(d, axis=1, keepdims=True)
    iota = lax.broadcasted_iota(jnp.int32, (ROWS, K), 1)
    ind_ref[...] = jnp.min(jnp.where(d == m, iota, K), axis=1, keepdims=True)


def _dist_argmax(xf, embed_t, x2, y2):
    grid = (BN // ROWS,)
    return pl.pallas_call(
        _dist_argmax_body,
        grid=grid,
        in_specs=[
            pl.BlockSpec((ROWS, DIM), lambda i: (i, 0)),
            pl.BlockSpec((DIM, K), lambda i: (0, 0)),
            pl.BlockSpec((ROWS, 1), lambda i: (i, 0)),
            pl.BlockSpec((1, K), lambda i: (0, 0)),
        ],
        out_specs=[
            pl.BlockSpec((ROWS, K), lambda i: (i, 0)),
            pl.BlockSpec((ROWS, 1), lambda i: (i, 0)),
        ],
        out_shape=[
            jax.ShapeDtypeStruct((BN, K), jnp.float32),
            jax.ShapeDtypeStruct((BN, 1), jnp.int32),
        ],
    )(xf, embed_t, x2, y2)


_NC, _NS = 2, 16
_NW = _NC * _NS
_BPW = BN // _NW
_CHUNK = 128
_NCHUNK = _BPW // _CHUNK


@functools.lru_cache(maxsize=None)
def _make_sc_gather():
    mesh = plsc.VectorSubcoreMesh(core_axis_name="c", subcore_axis_name="s")

    @functools.partial(
        pl.kernel, mesh=mesh,
        out_type=jax.ShapeDtypeStruct((BN, DIM), jnp.float32),
        scratch_types=[
            pltpu.VMEM((_NCHUNK, _CHUNK), jnp.int32),
            pltpu.VMEM((_CHUNK, DIM), jnp.float32),
            pltpu.SemaphoreType.DMA,
        ],
    )
    def sc_gather(idx_hbm, table_hbm, out_hbm, idx_v, rows_v, sem):
        wid = lax.axis_index("s") * _NC + lax.axis_index("c")
        pltpu.sync_copy(idx_hbm.at[wid], idx_v)
        base = wid * _BPW
        for c in range(_NCHUNK):
            pltpu.async_copy(table_hbm.at[idx_v.at[c]], rows_v, sem).wait()
            pltpu.sync_copy(rows_v, out_hbm.at[pl.ds(base + c * _CHUNK, _CHUNK)])

    return sc_gather


def kernel(x, embed):
    xf = x.reshape(BN, DIM)
    table = embed[0]
    embed_t = jnp.swapaxes(table, 0, 1)
    x2 = jnp.sum(xf * xf, axis=1, keepdims=True)
    y2 = jnp.sum(table * table, axis=-1)[None, :]
    dist2d, ind2d = _dist_argmax(xf, embed_t, x2, y2)
    ind = ind2d.reshape(BN)
    idx3 = ind.reshape(_NW, _NCHUNK, _CHUNK)
    quantize = _make_sc_gather()(idx3, table)
    return (
        quantize.reshape(B, N, DIM),
        ind.reshape(B, N),
        dist2d.reshape(1, B, N, K),
    )


# speedup vs baseline: 1.5273x; 1.0429x over previous
import functools

import jax
import jax.numpy as jnp
from jax import lax
from jax.experimental import pallas as pl
from jax.experimental.pallas import tpu as pltpu
from jax.experimental.pallas import tpu_sc as plsc


DIM = 256
K = 8192
B = 16
N = 1024
BN = B * N

ROWS = 512


def _dist_argmax_body(x_ref, et_ref, x2_ref, y2_ref, dist_ref, ind_ref):
    x = x_ref[...]
    et = et_ref[...]
    xy = jnp.dot(x, et, preferred_element_type=jnp.float32)
    dd = x2_ref[...] + y2_ref[...] + (-2.0) * xy
    d = -jnp.sqrt(jnp.clip(dd, 0.0, None))
    dist_ref[...] = d
    m = jnp.max(d, axis=1, keepdims=True)
    iota = lax.broadcasted_iota(jnp.int32, (ROWS, K), 1).astype(jnp.float32)
    ind_f = jnp.min(jnp.where(d == m, iota, float(K)), axis=1, keepdims=True)
    ind_ref[...] = ind_f.astype(jnp.int32)


def _dist_argmax(xf, embed_t, x2, y2):
    grid = (BN // ROWS,)
    return pl.pallas_call(
        _dist_argmax_body,
        grid=grid,
        in_specs=[
            pl.BlockSpec((ROWS, DIM), lambda i: (i, 0)),
            pl.BlockSpec((DIM, K), lambda i: (0, 0)),
            pl.BlockSpec((ROWS, 1), lambda i: (i, 0)),
            pl.BlockSpec((1, K), lambda i: (0, 0)),
        ],
        out_specs=[
            pl.BlockSpec((ROWS, K), lambda i: (i, 0)),
            pl.BlockSpec((ROWS, 1), lambda i: (i, 0)),
        ],
        out_shape=[
            jax.ShapeDtypeStruct((BN, K), jnp.float32),
            jax.ShapeDtypeStruct((BN, 1), jnp.int32),
        ],
    )(xf, embed_t, x2, y2)


_NC, _NS = 2, 16
_NW = _NC * _NS
_BPW = BN // _NW
_CHUNK = 128
_NCHUNK = _BPW // _CHUNK


@functools.lru_cache(maxsize=None)
def _make_sc_gather():
    mesh = plsc.VectorSubcoreMesh(core_axis_name="c", subcore_axis_name="s")

    @functools.partial(
        pl.kernel, mesh=mesh,
        out_type=jax.ShapeDtypeStruct((BN, DIM), jnp.float32),
        scratch_types=[
            pltpu.VMEM((_NCHUNK, _CHUNK), jnp.int32),
            pltpu.VMEM((_CHUNK, DIM), jnp.float32),
            pltpu.SemaphoreType.DMA,
        ],
    )
    def sc_gather(idx_hbm, table_hbm, out_hbm, idx_v, rows_v, sem):
        wid = lax.axis_index("s") * _NC + lax.axis_index("c")
        pltpu.sync_copy(idx_hbm.at[wid], idx_v)
        base = wid * _BPW
        for c in range(_NCHUNK):
            pltpu.async_copy(table_hbm.at[idx_v.at[c]], rows_v, sem).wait()
            pltpu.sync_copy(rows_v, out_hbm.at[pl.ds(base + c * _CHUNK, _CHUNK)])

    return sc_gather


def kernel(x, embed):
    xf = x.reshape(BN, DIM)
    table = embed[0]
    embed_t = jnp.swapaxes(table, 0, 1)
    x2 = jnp.sum(xf * xf, axis=1, keepdims=True)
    y2 = jnp.sum(table * table, axis=-1)[None, :]
    dist2d, ind2d = _dist_argmax(xf, embed_t, x2, y2)
    ind = ind2d.reshape(BN)
    idx3 = ind.reshape(_NW, _NCHUNK, _CHUNK)
    quantize = _make_sc_gather()(idx3, table)
    return (
        quantize.reshape(B, N, DIM),
        ind.reshape(B, N),
        dist2d.reshape(1, B, N, K),
    )
